# Initial kernel scaffold; baseline (speedup 1.0000x reference)
import jax
import jax.numpy as jnp
from jax.experimental import pallas as pl


def kernel(inputs, edge_matrix, mask, reversed_sess_item, sess_item, D, A, sess_len, embedding, a_0, a_1, a_2, a_3, adj_row, adj_col, adj_val):
    raise NotImplementedError("write your pallas kernel here")


# speedup vs baseline: 1.9017x; 1.9017x over previous
import functools

import jax
import jax.numpy as jnp
from jax import lax
from jax.experimental import pallas as pl
from jax.experimental.pallas import tpu as pltpu
from jax.experimental.pallas import tpu_sc as plsc

V = 50000
DIM = 100
DP = 128
F = 32
C = 4
B = 128
L = 40
E = 800000
NT = 16
NC = 2
VP = 50048
EPT = 50048
EPAD = EPT * NT
GB = 128
NBATCH = EPT // GB
RPT = VP // NT
ALPHA = 0.2


def _make_spmv(colmul: int, chunkmul: int):
    mesh = plsc.VectorSubcoreMesh(core_axis_name="c", subcore_axis_name="s")

    @functools.partial(
        pl.kernel,
        mesh=mesh,
        out_type=jax.ShapeDtypeStruct((C, VP, F), jnp.float32),
        compiler_params=pltpu.CompilerParams(use_tc_tiling_on_sc=False),
        scratch_types=[
            pltpu.VMEM((GB,), jnp.int32),
            pltpu.VMEM((GB,), jnp.int32),
            pltpu.VMEM((GB,), jnp.float32),
            pltpu.VMEM((GB,), jnp.int32),
            pltpu.VMEM((GB, F), jnp.float32),
            pltpu.VMEM_SHARED((VP, F), jnp.float32),
            pltpu.SemaphoreType.DMA,
        ],
    )
    def spmv(tab_hbm, row_hbm, col_hbm, val_hbm, zer_hbm, out_hbm,
             colb, rowb, valb, idxb, rows, acc, sem):
        cid = lax.axis_index("c")
        sid = lax.axis_index("s")
        ebase = sid * EPT

        for p in range(C // NC):
            chunk = p * NC + cid

            pltpu.sync_copy(zer_hbm.at[pl.ds(sid * RPT, RPT)],
                            acc.at[pl.ds(sid * RPT, RPT)])
            plsc.subcore_barrier()

            def batch(b, carry):
                off = ebase + b * GB
                pltpu.sync_copy(col_hbm.at[pl.ds(off, GB)], colb)
                pltpu.sync_copy(row_hbm.at[pl.ds(off, GB)], rowb)
                pltpu.sync_copy(val_hbm.at[pl.ds(off, GB)], valb)

                def idx_body(i, carry2):
                    cv = colb[pl.ds(i * 16, 16)]
                    idxb[pl.ds(i * 16, 16)] = cv * colmul + chunk * chunkmul
                    return carry2
                lax.fori_loop(0, GB // 16, idx_body, 0)

                pltpu.async_copy(tab_hbm.at[idxb], rows, sem).wait()

                def mul_body(k, carry2):
                    vv16 = valb[pl.ds(k * 16, 16)]
                    for j in range(16):
                        g = k * 16 + j
                        vv = jnp.full((16,), vv16[j], jnp.float32)
                        rows[g, pl.ds(0, 16)] = rows[g, pl.ds(0, 16)] * vv
                        rows[g, pl.ds(16, 16)] = rows[g, pl.ds(16, 16)] * vv
                    return carry2
                lax.fori_loop(0, GB // 16, mul_body, 0)

                pltpu.sync_copy(rows, acc.at[rowb], add=True)
                return carry
            lax.fori_loop(0, NBATCH, batch, 0)

            plsc.subcore_barrier()
            pltpu.sync_copy(acc.at[pl.ds(sid * RPT, RPT)],
                            out_hbm.at[chunk, pl.ds(sid * RPT, RPT)])
            plsc.subcore_barrier()

    return spmv


_spmv_layer1 = _make_spmv(colmul=C, chunkmul=1)
_spmv_layer2 = _make_spmv(colmul=1, chunkmul=VP)


NIDX = B * L
NW = NC * NT
IPW = NIDX // NW
SUB = 80
NSUB = IPW // SUB


def _make_gather():
    mesh = plsc.VectorSubcoreMesh(core_axis_name="c", subcore_axis_name="s")

    @functools.partial(
        pl.kernel,
        mesh=mesh,
        out_type=[
            jax.ShapeDtypeStruct((NIDX, DP), jnp.float32),
            jax.ShapeDtypeStruct((NIDX, DP), jnp.float32),
            jax.ShapeDtypeStruct((C, NIDX, F), jnp.float32),
            jax.ShapeDtypeStruct((C, NIDX, F), jnp.float32),
        ],
        compiler_params=pltpu.CompilerParams(use_tc_tiling_on_sc=False),
        scratch_types=[
            pltpu.VMEM((SUB,), jnp.int32),
            pltpu.VMEM((SUB,), jnp.int32),
            pltpu.VMEM((SUB, DP), jnp.float32),
            pltpu.VMEM((SUB, F), jnp.float32),
            pltpu.SemaphoreType.DMA,
        ],
    )
    def gat(emb_hbm, idxin_hbm, idxsess_hbm, l1_hbm, l2_hbm,
            h_hbm, ge_hbm, gl1_hbm, gl2_hbm,
            ib, ic, rwide, rnarrow, sem):
        cid = lax.axis_index("c")
        sid = lax.axis_index("s")
        wid = sid * NC + cid
        for s in range(NSUB):
            base = wid * IPW + s * SUB
            pltpu.sync_copy(idxin_hbm.at[pl.ds(base, SUB)], ib)
            pltpu.async_copy(emb_hbm.at[ib], rwide, sem).wait()
            pltpu.sync_copy(rwide, h_hbm.at[pl.ds(base, SUB)])
            pltpu.sync_copy(idxsess_hbm.at[pl.ds(base, SUB)], ib)
            pltpu.async_copy(emb_hbm.at[ib], rwide, sem).wait()
            pltpu.sync_copy(rwide, ge_hbm.at[pl.ds(base, SUB)])
            for c in range(C):
                def addoff(i, carry):
                    ic[pl.ds(i * 16, 16)] = ib[pl.ds(i * 16, 16)] + c * VP
                    return carry
                lax.fori_loop(0, SUB // 16, addoff, 0)
                pltpu.async_copy(l1_hbm.at[ic], rnarrow, sem).wait()
                pltpu.sync_copy(rnarrow, gl1_hbm.at[c, pl.ds(base, SUB)])
                pltpu.async_copy(l2_hbm.at[ic], rnarrow, sem).wait()
                pltpu.sync_copy(rnarrow, gl2_hbm.at[c, pl.ds(base, SUB)])

    return gat


_gather_rows = _make_gather()


def _leaky(x):
    return jnp.where(x >= 0, x, ALPHA * x)


def _attn_body(h_ref, em_ref, ge_ref, gl1_ref, gl2_ref, mf_ref, sl_ref, a4_ref,
               out_ref, sess0_ref):
    h = h_ref[0]
    em = em_ref[0]
    a4 = a4_ref[...]
    dn = (((1,), (1,)), ((), ()))
    e0 = _leaky(lax.dot_general(h * a4[0][None, :], h, dn))
    e1 = _leaky(lax.dot_general(h * a4[1][None, :], h, dn))
    e2 = _leaky(lax.dot_general(h * a4[2][None, :], h, dn))
    e3 = _leaky(lax.dot_general(h * a4[3][None, :], h, dn))
    big_neg = jnp.full_like(e0, -9e15)
    al = jnp.where(em == 1, e0, big_neg)
    al = jnp.where(em == 2, e1, al)
    al = jnp.where(em == 3, e2, al)
    al = jnp.where(em == 4, e3, al)
    al = al - jnp.max(al, axis=1, keepdims=True)
    al = jnp.exp(al)
    al = al / jnp.sum(al, axis=1, keepdims=True)
    intra = lax.dot_general(al, h, (((1,), (0,)), ((), ())))
    mf = mf_ref[0]
    seq = (ge_ref[0] + gl1_ref[0] + gl2_ref[0]) * (mf * (1.0 / 3.0))
    out_ref[0] = intra + seq
    seq1 = ge_ref[0] * mf
    sess0_ref[0] = jnp.sum(seq1, axis=0, keepdims=True) / sl_ref[0]


def _sess_body(s0_ref, d_ref, a_ref, out_ref):
    s0 = s0_ref[...]
    da = jnp.dot(d_ref[...], a_ref[...])
    s1 = jnp.dot(da, s0)
    s2 = jnp.dot(da, s1)
    out_ref[...] = (s0 + s1 + s2) * (1.0 / 3.0)


def kernel(inputs, edge_matrix, mask, reversed_sess_item, sess_item, D, A,
           sess_len, embedding, a_0, a_1, a_2, a_3, adj_row, adj_col, adj_val):
    del mask, reversed_sess_item

    emb128 = jnp.pad(embedding.astype(jnp.float32), ((0, 0), (0, DP - DIM)))
    embflat = emb128.reshape(V * C, F)

    pad = EPAD - E
    padi = jnp.arange(pad, dtype=jnp.int32) % V
    rowp = jnp.concatenate([adj_row.astype(jnp.int32), padi])
    colp = jnp.concatenate([adj_col.astype(jnp.int32), padi])
    valp = jnp.concatenate([adj_val.astype(jnp.float32),
                            jnp.zeros((pad,), jnp.float32)])

    zer = jnp.zeros((VP, F), jnp.float32)
    l1t = _spmv_layer1(embflat, rowp, colp, valp, zer)
    l1flat = l1t.reshape(C * VP, F)
    l2t = _spmv_layer2(l1flat, rowp, colp, valp, zer)
    l2flat = l2t.reshape(C * VP, F)

    idx_in = inputs.astype(jnp.int32).reshape(NIDX)
    si = sess_item.astype(jnp.int32).reshape(NIDX)
    idx_sess = jnp.maximum(si - 1, 0)
    h, ge, gl1, gl2 = _gather_rows(emb128, idx_in, idx_sess, l1flat, l2flat)

    hb = h.reshape(B, L, DP)
    geb = ge.reshape(B, L, DP)
    gl1b = gl1.transpose(1, 0, 2).reshape(B, L, DP)
    gl2b = gl2.transpose(1, 0, 2).reshape(B, L, DP)
    mf = (si > 0).astype(jnp.float32).reshape(B, L, 1)
    slr = sess_len.astype(jnp.float32).reshape(B, 1, 1)
    a4 = jnp.concatenate([a_0, a_1, a_2, a_3], axis=1).T.astype(jnp.float32)
    a4 = jnp.pad(a4, ((0, 0), (0, DP - DIM)))
    em = edge_matrix.astype(jnp.int32)

    out, sess0 = pl.pallas_call(
        _attn_body,
        grid=(B,),
        in_specs=[
            pl.BlockSpec((1, L, DP), lambda b: (b, 0, 0)),
            pl.BlockSpec((1, L, L), lambda b: (b, 0, 0)),
            pl.BlockSpec((1, L, DP), lambda b: (b, 0, 0)),
            pl.BlockSpec((1, L, DP), lambda b: (b, 0, 0)),
            pl.BlockSpec((1, L, DP), lambda b: (b, 0, 0)),
            pl.BlockSpec((1, L, 1), lambda b: (b, 0, 0)),
            pl.BlockSpec((1, 1, 1), lambda b: (b, 0, 0)),
            pl.BlockSpec((4, DP), lambda b: (0, 0)),
        ],
        out_specs=[
            pl.BlockSpec((1, L, DP), lambda b: (b, 0, 0)),
            pl.BlockSpec((1, 1, DP), lambda b: (b, 0, 0)),
        ],
        out_shape=[
            jax.ShapeDtypeStruct((B, L, DP), jnp.float32),
            jax.ShapeDtypeStruct((B, 1, DP), jnp.float32),
        ],
    )(hb, em, geb, gl1b, gl2b, mf, slr, a4)

    sess = pl.pallas_call(
        _sess_body,
        out_shape=jax.ShapeDtypeStruct((B, DP), jnp.float32),
    )(sess0.reshape(B, DP), D.astype(jnp.float32), A.astype(jnp.float32))

    return (out[:, :, :DIM], sess[:, :DIM])


# speedup vs baseline: 6.2886x; 3.3068x over previous
import functools

import jax
import jax.numpy as jnp
from jax import lax
from jax.experimental import pallas as pl
from jax.experimental.pallas import tpu as pltpu
from jax.experimental.pallas import tpu_sc as plsc

V = 50000
DIM = 100
DP = 128
F = 32
C = 4
B = 128
L = 40
E = 800000
NT = 16
NC = 2
VP = 50048
EPT = 51200
EPAD = EPT * NT
RPT = VP // NT
NQ = 20
QE = EPT // NQ
QR = QE // 128
SB = 256
NSB = QE // SB
KS = SB // 128
ALPHA = 0.2


def _make_spmv(colmul: int, chunkmul: int):
    mesh = plsc.VectorSubcoreMesh(core_axis_name="c", subcore_axis_name="s")

    @functools.partial(
        pl.kernel,
        mesh=mesh,
        out_type=jax.ShapeDtypeStruct((C, VP, F), jnp.float32),
        compiler_params=pltpu.CompilerParams(use_tc_tiling_on_sc=False),
        scratch_types=[
            pltpu.VMEM((QR, 128), jnp.int32),
            pltpu.VMEM((QR, 128), jnp.int32),
            pltpu.VMEM((QR, 128), jnp.float32),
            pltpu.VMEM((2, SB, F), jnp.float32),
            pltpu.VMEM_SHARED((VP, F), jnp.float32),
            pltpu.SemaphoreType.DMA((2,)),
            pltpu.SemaphoreType.DMA((2,)),
        ],
    )
    def spmv(tab_hbm, row_hbm, col_hbm, val_hbm, zer_hbm, out_hbm,
             idxq, rowq, valq, rows3, acc, gsem, ssem):
        cid = lax.axis_index("c")
        sid = lax.axis_index("s")
        rowbase = sid * (EPT // 128)

        for p in range(C // NC):
            chunk = p * NC + cid

            pltpu.sync_copy(zer_hbm.at[pl.ds(sid * RPT, RPT)],
                            acc.at[pl.ds(sid * RPT, RPT)])
            plsc.subcore_barrier()

            def quarter(q, carry):
                qoff = rowbase + q * QR
                pltpu.sync_copy(col_hbm.at[pl.ds(qoff, QR)], idxq)
                pltpu.sync_copy(row_hbm.at[pl.ds(qoff, QR)], rowq)
                pltpu.sync_copy(val_hbm.at[pl.ds(qoff, QR)], valq)

                def idx_body(r, carry2):
                    for j in range(8):
                        cv = idxq[r, pl.ds(j * 16, 16)]
                        idxq[r, pl.ds(j * 16, 16)] = (
                            cv * colmul + chunk * chunkmul)
                    return carry2
                lax.fori_loop(0, QR, idx_body, 0)

                for k in range(KS):
                    pltpu.async_copy(tab_hbm.at[idxq.at[k]],
                                     rows3.at[0, pl.ds(k * 128, 128)],
                                     gsem.at[0])

                def sb_body(b, carry2):
                    pi = lax.rem(b, 2)
                    pn = 1 - pi

                    @pl.when(b > 0)
                    def _():
                        pltpu.make_async_copy(
                            rows3.at[pn], acc.at[pl.ds(0, SB)],
                            ssem.at[pn]).wait()

                    @pl.when(b + 1 < NSB)
                    def _():
                        for k in range(KS):
                            pltpu.async_copy(
                                tab_hbm.at[idxq.at[(b + 1) * KS + k]],
                                rows3.at[pn, pl.ds(k * 128, 128)],
                                gsem.at[pn])

                    pltpu.make_async_copy(tab_hbm.at[pl.ds(0, SB)],
                                          rows3.at[pi], gsem.at[pi]).wait()

                    def mul_body(k2, carry3):
                        vr = b * KS + lax.div(k2, 8)
                        vo = lax.rem(k2, 8) * 16
                        vv16 = valq[vr, pl.ds(vo, 16)]
                        for j in range(16):
                            g = k2 * 16 + j
                            vv = jnp.full((16,), vv16[j], jnp.float32)
                            rows3[pi, g, pl.ds(0, 16)] = (
                                rows3[pi, g, pl.ds(0, 16)] * vv)
                            rows3[pi, g, pl.ds(16, 16)] = (
                                rows3[pi, g, pl.ds(16, 16)] * vv)
                        return carry3
                    lax.fori_loop(0, SB // 16, mul_body, 0)

                    for k in range(KS):
                        pltpu.async_copy(rows3.at[pi, pl.ds(k * 128, 128)],
                                         acc.at[rowq.at[b * KS + k]],
                                         ssem.at[pi], add=True)
                    return carry2
                lax.fori_loop(0, NSB, sb_body, 0)

                pltpu.make_async_copy(rows3.at[(NSB - 1) % 2],
                                      acc.at[pl.ds(0, SB)],
                                      ssem.at[(NSB - 1) % 2]).wait()
                return carry
            lax.fori_loop(0, NQ, quarter, 0)

            plsc.subcore_barrier()
            pltpu.sync_copy(acc.at[pl.ds(sid * RPT, RPT)],
                            out_hbm.at[chunk, pl.ds(sid * RPT, RPT)])
            plsc.subcore_barrier()

    return spmv


_spmv_layer1 = _make_spmv(colmul=C, chunkmul=1)
_spmv_layer2 = _make_spmv(colmul=1, chunkmul=VP)


NIDX = B * L
NW = NC * NT
IPW = NIDX // NW
SUB = 80
NSUB = IPW // SUB


def _make_gather():
    mesh = plsc.VectorSubcoreMesh(core_axis_name="c", subcore_axis_name="s")

    @functools.partial(
        pl.kernel,
        mesh=mesh,
        out_type=[
            jax.ShapeDtypeStruct((NIDX, DP), jnp.float32),
            jax.ShapeDtypeStruct((NIDX, DP), jnp.float32),
            jax.ShapeDtypeStruct((C, NIDX, F), jnp.float32),
            jax.ShapeDtypeStruct((C, NIDX, F), jnp.float32),
        ],
        compiler_params=pltpu.CompilerParams(use_tc_tiling_on_sc=False),
        scratch_types=[
            pltpu.VMEM((SUB,), jnp.int32),
            pltpu.VMEM((SUB,), jnp.int32),
            pltpu.VMEM((SUB, DP), jnp.float32),
            pltpu.VMEM((SUB, F), jnp.float32),
            pltpu.SemaphoreType.DMA,
        ],
    )
    def gat(emb_hbm, idxin_hbm, idxsess_hbm, l1_hbm, l2_hbm,
            h_hbm, ge_hbm, gl1_hbm, gl2_hbm,
            ib, ic, rwide, rnarrow, sem):
        cid = lax.axis_index("c")
        sid = lax.axis_index("s")
        wid = sid * NC + cid
        for s in range(NSUB):
            base = wid * IPW + s * SUB
            pltpu.sync_copy(idxin_hbm.at[pl.ds(base, SUB)], ib)
            pltpu.async_copy(emb_hbm.at[ib], rwide, sem).wait()
            pltpu.sync_copy(rwide, h_hbm.at[pl.ds(base, SUB)])
            pltpu.sync_copy(idxsess_hbm.at[pl.ds(base, SUB)], ib)
            pltpu.async_copy(emb_hbm.at[ib], rwide, sem).wait()
            pltpu.sync_copy(rwide, ge_hbm.at[pl.ds(base, SUB)])
            for c in range(C):
                def addoff(i, carry):
                    ic[pl.ds(i * 16, 16)] = ib[pl.ds(i * 16, 16)] + c * VP
                    return carry
                lax.fori_loop(0, SUB // 16, addoff, 0)
                pltpu.async_copy(l1_hbm.at[ic], rnarrow, sem).wait()
                pltpu.sync_copy(rnarrow, gl1_hbm.at[c, pl.ds(base, SUB)])
                pltpu.async_copy(l2_hbm.at[ic], rnarrow, sem).wait()
                pltpu.sync_copy(rnarrow, gl2_hbm.at[c, pl.ds(base, SUB)])

    return gat


_gather_rows = _make_gather()


def _leaky(x):
    return jnp.where(x >= 0, x, ALPHA * x)


def _attn_body(h_ref, em_ref, ge_ref, gl1_ref, gl2_ref, mf_ref, sl_ref, a4_ref,
               out_ref, sess0_ref):
    h = h_ref[0]
    em = em_ref[0]
    a4 = a4_ref[...]
    dn = (((1,), (1,)), ((), ()))
    e0 = _leaky(lax.dot_general(h * a4[0][None, :], h, dn))
    e1 = _leaky(lax.dot_general(h * a4[1][None, :], h, dn))
    e2 = _leaky(lax.dot_general(h * a4[2][None, :], h, dn))
    e3 = _leaky(lax.dot_general(h * a4[3][None, :], h, dn))
    big_neg = jnp.full_like(e0, -9e15)
    al = jnp.where(em == 1, e0, big_neg)
    al = jnp.where(em == 2, e1, al)
    al = jnp.where(em == 3, e2, al)
    al = jnp.where(em == 4, e3, al)
    al = al - jnp.max(al, axis=1, keepdims=True)
    al = jnp.exp(al)
    al = al / jnp.sum(al, axis=1, keepdims=True)
    intra = lax.dot_general(al, h, (((1,), (0,)), ((), ())))
    mf = mf_ref[0]
    seq = (ge_ref[0] + gl1_ref[0] + gl2_ref[0]) * (mf * (1.0 / 3.0))
    out_ref[0] = intra + seq
    seq1 = ge_ref[0] * mf
    sess0_ref[0] = jnp.sum(seq1, axis=0, keepdims=True) / sl_ref[0]


def _sess_body(s0_ref, d_ref, a_ref, out_ref):
    s0 = s0_ref[...]
    da = jnp.dot(d_ref[...], a_ref[...])
    s1 = jnp.dot(da, s0)
    s2 = jnp.dot(da, s1)
    out_ref[...] = (s0 + s1 + s2) * (1.0 / 3.0)


def kernel(inputs, edge_matrix, mask, reversed_sess_item, sess_item, D, A,
           sess_len, embedding, a_0, a_1, a_2, a_3, adj_row, adj_col, adj_val):
    del mask, reversed_sess_item

    emb128 = jnp.pad(embedding.astype(jnp.float32), ((0, 0), (0, DP - DIM)))
    embflat = emb128.reshape(V * C, F)

    pad = EPAD - E
    padi = jnp.arange(pad, dtype=jnp.int32) % V
    rowp = jnp.concatenate([adj_row.astype(jnp.int32), padi]).reshape(-1, 128)
    colp = jnp.concatenate([adj_col.astype(jnp.int32), padi]).reshape(-1, 128)
    valp = jnp.concatenate([adj_val.astype(jnp.float32),
                            jnp.zeros((pad,), jnp.float32)]).reshape(-1, 128)

    zer = jnp.zeros((VP, F), jnp.float32)
    l1t = _spmv_layer1(embflat, rowp, colp, valp, zer)
    l1flat = l1t.reshape(C * VP, F)
    l2t = _spmv_layer2(l1flat, rowp, colp, valp, zer)
    l2flat = l2t.reshape(C * VP, F)

    idx_in = inputs.astype(jnp.int32).reshape(NIDX)
    si = sess_item.astype(jnp.int32).reshape(NIDX)
    idx_sess = jnp.maximum(si - 1, 0)
    h, ge, gl1, gl2 = _gather_rows(emb128, idx_in, idx_sess, l1flat, l2flat)

    hb = h.reshape(B, L, DP)
    geb = ge.reshape(B, L, DP)
    gl1b = gl1.transpose(1, 0, 2).reshape(B, L, DP)
    gl2b = gl2.transpose(1, 0, 2).reshape(B, L, DP)
    mf = (si > 0).astype(jnp.float32).reshape(B, L, 1)
    slr = sess_len.astype(jnp.float32).reshape(B, 1, 1)
    a4 = jnp.concatenate([a_0, a_1, a_2, a_3], axis=1).T.astype(jnp.float32)
    a4 = jnp.pad(a4, ((0, 0), (0, DP - DIM)))
    em = edge_matrix.astype(jnp.int32)

    out, sess0 = pl.pallas_call(
        _attn_body,
        grid=(B,),
        in_specs=[
            pl.BlockSpec((1, L, DP), lambda b: (b, 0, 0)),
            pl.BlockSpec((1, L, L), lambda b: (b, 0, 0)),
            pl.BlockSpec((1, L, DP), lambda b: (b, 0, 0)),
            pl.BlockSpec((1, L, DP), lambda b: (b, 0, 0)),
            pl.BlockSpec((1, L, DP), lambda b: (b, 0, 0)),
            pl.BlockSpec((1, L, 1), lambda b: (b, 0, 0)),
            pl.BlockSpec((1, 1, 1), lambda b: (b, 0, 0)),
            pl.BlockSpec((4, DP), lambda b: (0, 0)),
        ],
        out_specs=[
            pl.BlockSpec((1, L, DP), lambda b: (b, 0, 0)),
            pl.BlockSpec((1, 1, DP), lambda b: (b, 0, 0)),
        ],
        out_shape=[
            jax.ShapeDtypeStruct((B, L, DP), jnp.float32),
            jax.ShapeDtypeStruct((B, 1, DP), jnp.float32),
        ],
    )(hb, em, geb, gl1b, gl2b, mf, slr, a4)

    sess = pl.pallas_call(
        _sess_body,
        out_shape=jax.ShapeDtypeStruct((B, DP), jnp.float32),
    )(sess0.reshape(B, DP), D.astype(jnp.float32), A.astype(jnp.float32))

    return (out[:, :, :DIM], sess[:, :DIM])


# speedup vs baseline: 7.7755x; 1.2364x over previous
import functools

import jax
import jax.numpy as jnp
from jax import lax
from jax.experimental import pallas as pl
from jax.experimental.pallas import tpu as pltpu
from jax.experimental.pallas import tpu_sc as plsc

V = 50000
DIM = 100
DP = 128
F = 32
C = 4
B = 128
L = 40
E = 800000
NT = 16
NC = 2
VP = 50048
EPT = 51200
EPAD = EPT * NT
RPT = VP // NT
NQ = 4
QE = EPT // NQ
QR = QE // 128
SB = 512
NSB = QE // SB
KS = SB // 128
ALPHA = 0.2


def _make_spmv(colmul: int, chunkmul: int):
    mesh = plsc.VectorSubcoreMesh(core_axis_name="c", subcore_axis_name="s")

    @functools.partial(
        pl.kernel,
        mesh=mesh,
        out_type=jax.ShapeDtypeStruct((C, VP, F), jnp.bfloat16),
        compiler_params=pltpu.CompilerParams(use_tc_tiling_on_sc=False),
        scratch_types=[
            pltpu.VMEM((QR, 128), jnp.int32),
            pltpu.VMEM((QR, 128), jnp.int32),
            pltpu.VMEM((QR, 128), jnp.float32),
            pltpu.VMEM((2, SB, F), jnp.bfloat16),
            pltpu.VMEM_SHARED((VP, F), jnp.bfloat16),
            pltpu.SemaphoreType.DMA((2,)),
            pltpu.SemaphoreType.DMA((2,)),
        ],
    )
    def spmv(tab_hbm, row_hbm, col_hbm, val_hbm, zer_hbm, out_hbm,
             idxq, rowq, valq, rows3, acc, gsem, ssem):
        cid = lax.axis_index("c")
        sid = lax.axis_index("s")
        rowbase = sid * (EPT // 128)

        for p in range(C // NC):
            chunk = p * NC + cid

            pltpu.sync_copy(zer_hbm.at[pl.ds(sid * RPT, RPT)],
                            acc.at[pl.ds(sid * RPT, RPT)])
            plsc.subcore_barrier()

            def quarter(q, carry):
                qoff = rowbase + q * QR
                pltpu.sync_copy(col_hbm.at[pl.ds(qoff, QR)], idxq)
                pltpu.sync_copy(row_hbm.at[pl.ds(qoff, QR)], rowq)
                pltpu.sync_copy(val_hbm.at[pl.ds(qoff, QR)], valq)

                def idx_body(r, carry2):
                    for j in range(8):
                        cv = idxq[r, pl.ds(j * 16, 16)]
                        idxq[r, pl.ds(j * 16, 16)] = (
                            cv * colmul + chunk * chunkmul)
                    return carry2
                lax.fori_loop(0, QR, idx_body, 0)

                for k in range(KS):
                    pltpu.async_copy(tab_hbm.at[idxq.at[k]],
                                     rows3.at[0, pl.ds(k * 128, 128)],
                                     gsem.at[0])

                def sb_body(b, carry2):
                    pi = lax.rem(b, 2)
                    pn = 1 - pi

                    @pl.when(b > 0)
                    def _():
                        pltpu.make_async_copy(
                            rows3.at[pn], acc.at[pl.ds(0, SB)],
                            ssem.at[pn]).wait()

                    @pl.when(b + 1 < NSB)
                    def _():
                        for k in range(KS):
                            pltpu.async_copy(
                                tab_hbm.at[idxq.at[(b + 1) * KS + k]],
                                rows3.at[pn, pl.ds(k * 128, 128)],
                                gsem.at[pn])

                    pltpu.make_async_copy(tab_hbm.at[pl.ds(0, SB)],
                                          rows3.at[pi], gsem.at[pi]).wait()

                    def mul_body(k2, carry3):
                        vr = b * KS + lax.div(k2, 8)
                        vo = lax.rem(k2, 8) * 16
                        vv16 = valq[vr, pl.ds(vo, 16)]
                        for j in range(16):
                            g = k2 * 16 + j
                            vv = jnp.full((32,), vv16[j], jnp.float32
                                          ).astype(jnp.bfloat16)
                            rows3[pi, g, pl.ds(0, 32)] = (
                                rows3[pi, g, pl.ds(0, 32)] * vv)
                        return carry3
                    lax.fori_loop(0, SB // 16, mul_body, 0)

                    for k in range(KS):
                        pltpu.async_copy(rows3.at[pi, pl.ds(k * 128, 128)],
                                         acc.at[rowq.at[b * KS + k]],
                                         ssem.at[pi], add=True)
                    return carry2
                lax.fori_loop(0, NSB, sb_body, 0)

                pltpu.make_async_copy(rows3.at[(NSB - 1) % 2],
                                      acc.at[pl.ds(0, SB)],
                                      ssem.at[(NSB - 1) % 2]).wait()
                return carry
            lax.fori_loop(0, NQ, quarter, 0)

            plsc.subcore_barrier()
            pltpu.sync_copy(acc.at[pl.ds(sid * RPT, RPT)],
                            out_hbm.at[chunk, pl.ds(sid * RPT, RPT)])
            plsc.subcore_barrier()

    return spmv


_spmv_layer1 = _make_spmv(colmul=C, chunkmul=1)
_spmv_layer2 = _make_spmv(colmul=1, chunkmul=VP)


NIDX = B * L
NW = NC * NT
IPW = NIDX // NW
SUB = 80
NSUB = IPW // SUB


def _make_gather():
    mesh = plsc.VectorSubcoreMesh(core_axis_name="c", subcore_axis_name="s")

    @functools.partial(
        pl.kernel,
        mesh=mesh,
        out_type=[
            jax.ShapeDtypeStruct((NIDX, DP), jnp.float32),
            jax.ShapeDtypeStruct((NIDX, DP), jnp.float32),
            jax.ShapeDtypeStruct((C, NIDX, F), jnp.bfloat16),
            jax.ShapeDtypeStruct((C, NIDX, F), jnp.bfloat16),
        ],
        compiler_params=pltpu.CompilerParams(use_tc_tiling_on_sc=False),
        scratch_types=[
            pltpu.VMEM((SUB,), jnp.int32),
            pltpu.VMEM((SUB,), jnp.int32),
            pltpu.VMEM((SUB, DP), jnp.float32),
            pltpu.VMEM((SUB, F), jnp.bfloat16),
            pltpu.SemaphoreType.DMA,
        ],
    )
    def gat(emb_hbm, idxin_hbm, idxsess_hbm, l1_hbm, l2_hbm,
            h_hbm, ge_hbm, gl1_hbm, gl2_hbm,
            ib, ic, rwide, rnarrow, sem):
        cid = lax.axis_index("c")
        sid = lax.axis_index("s")
        wid = sid * NC + cid
        for s in range(NSUB):
            base = wid * IPW + s * SUB
            pltpu.sync_copy(idxin_hbm.at[pl.ds(base, SUB)], ib)
            pltpu.async_copy(emb_hbm.at[ib], rwide, sem).wait()
            pltpu.sync_copy(rwide, h_hbm.at[pl.ds(base, SUB)])
            pltpu.sync_copy(idxsess_hbm.at[pl.ds(base, SUB)], ib)
            pltpu.async_copy(emb_hbm.at[ib], rwide, sem).wait()
            pltpu.sync_copy(rwide, ge_hbm.at[pl.ds(base, SUB)])
            for c in range(C):
                def addoff(i, carry):
                    ic[pl.ds(i * 16, 16)] = ib[pl.ds(i * 16, 16)] + c * VP
                    return carry
                lax.fori_loop(0, SUB // 16, addoff, 0)
                pltpu.async_copy(l1_hbm.at[ic], rnarrow, sem).wait()
                pltpu.sync_copy(rnarrow, gl1_hbm.at[c, pl.ds(base, SUB)])
                pltpu.async_copy(l2_hbm.at[ic], rnarrow, sem).wait()
                pltpu.sync_copy(rnarrow, gl2_hbm.at[c, pl.ds(base, SUB)])

    return gat


_gather_rows = _make_gather()


def _leaky(x):
    return jnp.where(x >= 0, x, ALPHA * x)


def _attn_body(h_ref, em_ref, ge_ref, gl1_ref, gl2_ref, mf_ref, sl_ref, a4_ref,
               out_ref, sess0_ref):
    h = h_ref[0]
    em = em_ref[0]
    a4 = a4_ref[...]
    dn = (((1,), (1,)), ((), ()))
    e0 = _leaky(lax.dot_general(h * a4[0][None, :], h, dn))
    e1 = _leaky(lax.dot_general(h * a4[1][None, :], h, dn))
    e2 = _leaky(lax.dot_general(h * a4[2][None, :], h, dn))
    e3 = _leaky(lax.dot_general(h * a4[3][None, :], h, dn))
    big_neg = jnp.full_like(e0, -9e15)
    al = jnp.where(em == 1, e0, big_neg)
    al = jnp.where(em == 2, e1, al)
    al = jnp.where(em == 3, e2, al)
    al = jnp.where(em == 4, e3, al)
    al = al - jnp.max(al, axis=1, keepdims=True)
    al = jnp.exp(al)
    al = al / jnp.sum(al, axis=1, keepdims=True)
    intra = lax.dot_general(al, h, (((1,), (0,)), ((), ())))
    mf = mf_ref[0]
    seq = (ge_ref[0] + gl1_ref[0] + gl2_ref[0]) * (mf * (1.0 / 3.0))
    out_ref[0] = intra + seq
    seq1 = ge_ref[0] * mf
    sess0_ref[0] = jnp.sum(seq1, axis=0, keepdims=True) / sl_ref[0]


def _sess_body(s0_ref, d_ref, a_ref, out_ref):
    s0 = s0_ref[...]
    da = jnp.dot(d_ref[...], a_ref[...])
    s1 = jnp.dot(da, s0)
    s2 = jnp.dot(da, s1)
    out_ref[...] = (s0 + s1 + s2) * (1.0 / 3.0)


def kernel(inputs, edge_matrix, mask, reversed_sess_item, sess_item, D, A,
           sess_len, embedding, a_0, a_1, a_2, a_3, adj_row, adj_col, adj_val):
    del mask, reversed_sess_item

    emb128 = jnp.pad(embedding.astype(jnp.float32), ((0, 0), (0, DP - DIM)))
    embflat = emb128.astype(jnp.bfloat16).reshape(V * C, F)

    pad = EPAD - E
    padi = jnp.arange(pad, dtype=jnp.int32) % V
    rowp = jnp.concatenate([adj_row.astype(jnp.int32), padi]).reshape(-1, 128)
    colp = jnp.concatenate([adj_col.astype(jnp.int32), padi]).reshape(-1, 128)
    valp = jnp.concatenate([adj_val.astype(jnp.float32),
                            jnp.zeros((pad,), jnp.float32)]).reshape(-1, 128)

    zer = jnp.zeros((VP, F), jnp.bfloat16)
    l1t = _spmv_layer1(embflat, rowp, colp, valp, zer)
    l1flat = l1t.reshape(C * VP, F)
    l2t = _spmv_layer2(l1flat, rowp, colp, valp, zer)
    l2flat = l2t.reshape(C * VP, F)

    idx_in = inputs.astype(jnp.int32).reshape(NIDX)
    si = sess_item.astype(jnp.int32).reshape(NIDX)
    idx_sess = jnp.maximum(si - 1, 0)
    h, ge, gl1, gl2 = _gather_rows(emb128, idx_in, idx_sess, l1flat, l2flat)

    hb = h.reshape(B, L, DP)
    geb = ge.reshape(B, L, DP)
    gl1b = gl1.astype(jnp.float32).transpose(1, 0, 2).reshape(B, L, DP)
    gl2b = gl2.astype(jnp.float32).transpose(1, 0, 2).reshape(B, L, DP)
    mf = (si > 0).astype(jnp.float32).reshape(B, L, 1)
    slr = sess_len.astype(jnp.float32).reshape(B, 1, 1)
    a4 = jnp.concatenate([a_0, a_1, a_2, a_3], axis=1).T.astype(jnp.float32)
    a4 = jnp.pad(a4, ((0, 0), (0, DP - DIM)))
    em = edge_matrix.astype(jnp.int32)

    out, sess0 = pl.pallas_call(
        _attn_body,
        grid=(B,),
        in_specs=[
            pl.BlockSpec((1, L, DP), lambda b: (b, 0, 0)),
            pl.BlockSpec((1, L, L), lambda b: (b, 0, 0)),
            pl.BlockSpec((1, L, DP), lambda b: (b, 0, 0)),
            pl.BlockSpec((1, L, DP), lambda b: (b, 0, 0)),
            pl.BlockSpec((1, L, DP), lambda b: (b, 0, 0)),
            pl.BlockSpec((1, L, 1), lambda b: (b, 0, 0)),
            pl.BlockSpec((1, 1, 1), lambda b: (b, 0, 0)),
            pl.BlockSpec((4, DP), lambda b: (0, 0)),
        ],
        out_specs=[
            pl.BlockSpec((1, L, DP), lambda b: (b, 0, 0)),
            pl.BlockSpec((1, 1, DP), lambda b: (b, 0, 0)),
        ],
        out_shape=[
            jax.ShapeDtypeStruct((B, L, DP), jnp.float32),
            jax.ShapeDtypeStruct((B, 1, DP), jnp.float32),
        ],
    )(hb, em, geb, gl1b, gl2b, mf, slr, a4)

    sess = pl.pallas_call(
        _sess_body,
        out_shape=jax.ShapeDtypeStruct((B, DP), jnp.float32),
    )(sess0.reshape(B, DP), D.astype(jnp.float32), A.astype(jnp.float32))

    return (out[:, :, :DIM], sess[:, :DIM])


# speedup vs baseline: 9.3166x; 1.1982x over previous
import functools

import jax
import jax.numpy as jnp
from jax import lax
from jax.experimental import pallas as pl
from jax.experimental.pallas import tpu as pltpu
from jax.experimental.pallas import tpu_sc as plsc

V = 50000
DIM = 100
DP = 128
F = 32
C = 4
B = 128
L = 40
E = 800000
NT = 16
NC = 2
VP = 50048
EPT = 51200
EPAD = EPT * NT
RPT = VP // NT
NQ = 4
QE = EPT // NQ
QR = QE // 128
SB = 512
NSB = QE // SB
KS = SB // 128
ALPHA = 0.2


def _make_spmv(colmul: int, chunkmul: int):
    mesh = plsc.VectorSubcoreMesh(core_axis_name="c", subcore_axis_name="s")

    @functools.partial(
        pl.kernel,
        mesh=mesh,
        out_type=jax.ShapeDtypeStruct((C * VP, F), jnp.bfloat16),
        compiler_params=pltpu.CompilerParams(use_tc_tiling_on_sc=False),
        scratch_types=[
            pltpu.VMEM((QR, 128), jnp.int32),
            pltpu.VMEM((QR, 128), jnp.int32),
            pltpu.VMEM((QR, 128), jnp.float32),
            pltpu.VMEM((2, SB, F), jnp.bfloat16),
            pltpu.VMEM_SHARED((VP, F), jnp.bfloat16),
            pltpu.SemaphoreType.DMA((2,)),
            pltpu.SemaphoreType.DMA((2,)),
        ],
    )
    def spmv(tab_hbm, row_hbm, col_hbm, val_hbm, zer_hbm, out_hbm,
             idxq, rowq, valq, rows3, acc, gsem, ssem):
        cid = lax.axis_index("c")
        sid = lax.axis_index("s")
        rowbase = sid * (EPT // 128)

        for p in range(C // NC):
            chunk = p * NC + cid

            pltpu.sync_copy(zer_hbm.at[pl.ds(sid * RPT, RPT)],
                            acc.at[pl.ds(sid * RPT, RPT)])
            plsc.subcore_barrier()

            def quarter(q, carry):
                qoff = rowbase + q * QR
                pltpu.sync_copy(col_hbm.at[pl.ds(qoff, QR)], idxq)
                pltpu.sync_copy(row_hbm.at[pl.ds(qoff, QR)], rowq)
                pltpu.sync_copy(val_hbm.at[pl.ds(qoff, QR)], valq)

                def idx_body(r, carry2):
                    for j in range(8):
                        cv = idxq[r, pl.ds(j * 16, 16)]
                        idxq[r, pl.ds(j * 16, 16)] = (
                            cv * colmul + chunk * chunkmul)
                    return carry2
                lax.fori_loop(0, QR, idx_body, 0)

                for k in range(KS):
                    pltpu.async_copy(tab_hbm.at[idxq.at[k]],
                                     rows3.at[0, pl.ds(k * 128, 128)],
                                     gsem.at[0])

                def sb_body(b, carry2):
                    pi = lax.rem(b, 2)
                    pn = 1 - pi

                    @pl.when(b > 0)
                    def _():
                        pltpu.make_async_copy(
                            rows3.at[pn], acc.at[pl.ds(0, SB)],
                            ssem.at[pn]).wait()

                    @pl.when(b + 1 < NSB)
                    def _():
                        for k in range(KS):
                            pltpu.async_copy(
                                tab_hbm.at[idxq.at[(b + 1) * KS + k]],
                                rows3.at[pn, pl.ds(k * 128, 128)],
                                gsem.at[pn])

                    pltpu.make_async_copy(tab_hbm.at[pl.ds(0, SB)],
                                          rows3.at[pi], gsem.at[pi]).wait()

                    def mul_body(k2, carry3):
                        vr = b * KS + lax.div(k2, 8)
                        vo = lax.rem(k2, 8) * 16
                        vv16 = valq[vr, pl.ds(vo, 16)]
                        for j in range(16):
                            g = k2 * 16 + j
                            vv = jnp.full((32,), vv16[j], jnp.float32
                                          ).astype(jnp.bfloat16)
                            rows3[pi, g, pl.ds(0, 32)] = (
                                rows3[pi, g, pl.ds(0, 32)] * vv)
                        return carry3
                    lax.fori_loop(0, SB // 16, mul_body, 0)

                    for k in range(KS):
                        pltpu.async_copy(rows3.at[pi, pl.ds(k * 128, 128)],
                                         acc.at[rowq.at[b * KS + k]],
                                         ssem.at[pi], add=True)
                    return carry2
                lax.fori_loop(0, NSB, sb_body, 0)

                pltpu.make_async_copy(rows3.at[(NSB - 1) % 2],
                                      acc.at[pl.ds(0, SB)],
                                      ssem.at[(NSB - 1) % 2]).wait()
                return carry
            lax.fori_loop(0, NQ, quarter, 0)

            plsc.subcore_barrier()
            pltpu.sync_copy(acc.at[pl.ds(sid * RPT, RPT)],
                            out_hbm.at[pl.ds(chunk * VP + sid * RPT, RPT)])
            plsc.subcore_barrier()

    return spmv


_spmv_layer1 = _make_spmv(colmul=C, chunkmul=1)
_spmv_layer2 = _make_spmv(colmul=1, chunkmul=VP)


NIDX = B * L
NW = NC * NT
IPW = NIDX // NW
SUB = 80
NSUB = IPW // SUB


def _make_gather():
    mesh = plsc.VectorSubcoreMesh(core_axis_name="c", subcore_axis_name="s")

    @functools.partial(
        pl.kernel,
        mesh=mesh,
        out_type=[
            jax.ShapeDtypeStruct((NIDX, DP), jnp.float32),
            jax.ShapeDtypeStruct((NIDX, DP), jnp.float32),
            jax.ShapeDtypeStruct((C, NIDX, F), jnp.bfloat16),
            jax.ShapeDtypeStruct((C, NIDX, F), jnp.bfloat16),
        ],
        compiler_params=pltpu.CompilerParams(use_tc_tiling_on_sc=False),
        scratch_types=[
            pltpu.VMEM((SUB,), jnp.int32),
            pltpu.VMEM((SUB,), jnp.int32),
            pltpu.VMEM((SUB, DP), jnp.float32),
            pltpu.VMEM((SUB, F), jnp.bfloat16),
            pltpu.SemaphoreType.DMA,
        ],
    )
    def gat(emb_hbm, idxin_hbm, idxsess_hbm, l1_hbm, l2_hbm,
            h_hbm, ge_hbm, gl1_hbm, gl2_hbm,
            ib, ic, rwide, rnarrow, sem):
        cid = lax.axis_index("c")
        sid = lax.axis_index("s")
        wid = sid * NC + cid
        for s in range(NSUB):
            base = wid * IPW + s * SUB
            pltpu.sync_copy(idxin_hbm.at[pl.ds(base, SUB)], ib)
            pltpu.async_copy(emb_hbm.at[ib], rwide, sem).wait()
            pltpu.sync_copy(rwide, h_hbm.at[pl.ds(base, SUB)])
            pltpu.sync_copy(idxsess_hbm.at[pl.ds(base, SUB)], ib)
            pltpu.async_copy(emb_hbm.at[ib], rwide, sem).wait()
            pltpu.sync_copy(rwide, ge_hbm.at[pl.ds(base, SUB)])
            for c in range(C):
                def addoff(i, carry):
                    ic[pl.ds(i * 16, 16)] = ib[pl.ds(i * 16, 16)] + c * VP
                    return carry
                lax.fori_loop(0, SUB // 16, addoff, 0)
                pltpu.async_copy(l1_hbm.at[ic], rnarrow, sem).wait()
                pltpu.sync_copy(rnarrow, gl1_hbm.at[c, pl.ds(base, SUB)])
                pltpu.async_copy(l2_hbm.at[ic], rnarrow, sem).wait()
                pltpu.sync_copy(rnarrow, gl2_hbm.at[c, pl.ds(base, SUB)])

    return gat


_gather_rows = _make_gather()


def _leaky(x):
    return jnp.where(x >= 0, x, ALPHA * x)


GS = 8


def _attn_body(h_ref, em_ref, ge_ref, gl1_ref, gl2_ref, mf_ref, sl_ref, a4_ref,
               out_ref, sess0_ref):
    a4 = a4_ref[...]
    dn = (((1,), (1,)), ((), ()))
    for i in range(GS):
        h = h_ref[i]
        em = em_ref[i]
        ha = jnp.concatenate(
            [h * a4[0], h * a4[1], h * a4[2], h * a4[3]], axis=0)
        e = _leaky(lax.dot_general(ha, h, dn))
        big_neg = jnp.full((L, L), -9e15, jnp.float32)
        al = jnp.where(em == 1, e[0 * L:1 * L], big_neg)
        al = jnp.where(em == 2, e[1 * L:2 * L], al)
        al = jnp.where(em == 3, e[2 * L:3 * L], al)
        al = jnp.where(em == 4, e[3 * L:4 * L], al)
        al = al - jnp.max(al, axis=1, keepdims=True)
        al = jnp.exp(al)
        al = al / jnp.sum(al, axis=1, keepdims=True)
        intra = lax.dot_general(al, h, (((1,), (0,)), ((), ())))
        mf = mf_ref[i]
        seq = (ge_ref[i] + gl1_ref[i] + gl2_ref[i]) * (mf * (1.0 / 3.0))
        out_ref[i] = intra + seq
        seq1 = ge_ref[i] * mf
        sess0_ref[i] = jnp.sum(seq1, axis=0, keepdims=True) / sl_ref[i]


def _sess_body(s0_ref, d_ref, a_ref, out_ref):
    s0 = s0_ref[...]
    da = jnp.dot(d_ref[...], a_ref[...])
    s1 = jnp.dot(da, s0)
    s2 = jnp.dot(da, s1)
    out_ref[...] = (s0 + s1 + s2) * (1.0 / 3.0)


def kernel(inputs, edge_matrix, mask, reversed_sess_item, sess_item, D, A,
           sess_len, embedding, a_0, a_1, a_2, a_3, adj_row, adj_col, adj_val):
    del mask, reversed_sess_item

    emb128 = jnp.pad(embedding.astype(jnp.float32), ((0, 0), (0, DP - DIM)))
    embflat = emb128.astype(jnp.bfloat16).reshape(V * C, F)

    pad = EPAD - E
    padi = jnp.arange(pad, dtype=jnp.int32) % V
    rowp = jnp.concatenate([adj_row.astype(jnp.int32), padi]).reshape(-1, 128)
    colp = jnp.concatenate([adj_col.astype(jnp.int32), padi]).reshape(-1, 128)
    valp = jnp.concatenate([adj_val.astype(jnp.float32),
                            jnp.zeros((pad,), jnp.float32)]).reshape(-1, 128)

    zer = jnp.zeros((VP, F), jnp.bfloat16)
    l1flat = _spmv_layer1(embflat, rowp, colp, valp, zer)
    l2flat = _spmv_layer2(l1flat, rowp, colp, valp, zer)

    idx_in = inputs.astype(jnp.int32).reshape(NIDX)
    si = sess_item.astype(jnp.int32).reshape(NIDX)
    idx_sess = jnp.maximum(si - 1, 0)
    h, ge, gl1, gl2 = _gather_rows(emb128, idx_in, idx_sess, l1flat, l2flat)

    hb = h.reshape(B, L, DP)
    geb = ge.reshape(B, L, DP)
    gl1b = gl1.astype(jnp.float32).transpose(1, 0, 2).reshape(B, L, DP)
    gl2b = gl2.astype(jnp.float32).transpose(1, 0, 2).reshape(B, L, DP)
    mf = (si > 0).astype(jnp.float32).reshape(B, L, 1)
    slr = sess_len.astype(jnp.float32).reshape(B, 1, 1)
    a4 = jnp.concatenate([a_0, a_1, a_2, a_3], axis=1).T.astype(jnp.float32)
    a4 = jnp.pad(a4, ((0, 0), (0, DP - DIM)))
    em = edge_matrix.astype(jnp.int32)

    out, sess0 = pl.pallas_call(
        _attn_body,
        grid=(B // GS,),
        in_specs=[
            pl.BlockSpec((GS, L, DP), lambda b: (b, 0, 0)),
            pl.BlockSpec((GS, L, L), lambda b: (b, 0, 0)),
            pl.BlockSpec((GS, L, DP), lambda b: (b, 0, 0)),
            pl.BlockSpec((GS, L, DP), lambda b: (b, 0, 0)),
            pl.BlockSpec((GS, L, DP), lambda b: (b, 0, 0)),
            pl.BlockSpec((GS, L, 1), lambda b: (b, 0, 0)),
            pl.BlockSpec((GS, 1, 1), lambda b: (b, 0, 0)),
            pl.BlockSpec((4, DP), lambda b: (0, 0)),
        ],
        out_specs=[
            pl.BlockSpec((GS, L, DP), lambda b: (b, 0, 0)),
            pl.BlockSpec((GS, 1, DP), lambda b: (b, 0, 0)),
        ],
        out_shape=[
            jax.ShapeDtypeStruct((B, L, DP), jnp.float32),
            jax.ShapeDtypeStruct((B, 1, DP), jnp.float32),
        ],
    )(hb, em, geb, gl1b, gl2b, mf, slr, a4)

    sess = pl.pallas_call(
        _sess_body,
        out_shape=jax.ShapeDtypeStruct((B, DP), jnp.float32),
    )(sess0.reshape(B, DP), D.astype(jnp.float32), A.astype(jnp.float32))

    return (out[:, :, :DIM], sess[:, :DIM])


# speedup vs baseline: 9.9262x; 1.0654x over previous
import functools

import jax
import jax.numpy as jnp
from jax import lax
from jax.experimental import pallas as pl
from jax.experimental.pallas import tpu as pltpu
from jax.experimental.pallas import tpu_sc as plsc

V = 50000
DIM = 100
DP = 128
F = 64
C = 2
B = 128
L = 40
E = 800000
NT = 16
NC = 2
VP = 50048
EPT = 51200
EPAD = EPT * NT
RPT = VP // NT
NQ = 20
QE = EPT // NQ
QR = QE // 128
SB = 256
NSB = QE // SB
KS = SB // 128
ALPHA = 0.2


def _make_spmv(colmul: int, chunkmul: int):
    mesh = plsc.VectorSubcoreMesh(core_axis_name="c", subcore_axis_name="s")

    @functools.partial(
        pl.kernel,
        mesh=mesh,
        out_type=jax.ShapeDtypeStruct((C * VP, F), jnp.bfloat16),
        compiler_params=pltpu.CompilerParams(use_tc_tiling_on_sc=False),
        scratch_types=[
            pltpu.VMEM((QR, 128), jnp.int32),
            pltpu.VMEM((QR, 128), jnp.int32),
            pltpu.VMEM((QR, 128), jnp.float32),
            pltpu.VMEM((2, SB, F), jnp.bfloat16),
            pltpu.VMEM_SHARED((VP, F), jnp.bfloat16),
            pltpu.SemaphoreType.DMA((2,)),
            pltpu.SemaphoreType.DMA((2,)),
        ],
    )
    def spmv(tab_hbm, row_hbm, col_hbm, val_hbm, zer_hbm, out_hbm,
             idxq, rowq, valq, rows3, acc, gsem, ssem):
        cid = lax.axis_index("c")
        sid = lax.axis_index("s")
        rowbase = sid * (EPT // 128)

        for p in range(C // NC):
            chunk = p * NC + cid

            pltpu.sync_copy(zer_hbm.at[pl.ds(sid * RPT, RPT)],
                            acc.at[pl.ds(sid * RPT, RPT)])
            plsc.subcore_barrier()

            def quarter(q, carry):
                qoff = rowbase + q * QR
                pltpu.sync_copy(col_hbm.at[pl.ds(qoff, QR)], idxq)
                pltpu.sync_copy(row_hbm.at[pl.ds(qoff, QR)], rowq)
                pltpu.sync_copy(val_hbm.at[pl.ds(qoff, QR)], valq)

                def idx_body(r, carry2):
                    for j in range(8):
                        cv = idxq[r, pl.ds(j * 16, 16)]
                        idxq[r, pl.ds(j * 16, 16)] = (
                            cv * colmul + chunk * chunkmul)
                    return carry2
                lax.fori_loop(0, QR, idx_body, 0)

                for k in range(KS):
                    pltpu.async_copy(tab_hbm.at[idxq.at[k]],
                                     rows3.at[0, pl.ds(k * 128, 128)],
                                     gsem.at[0])

                def sb_body(b, carry2):
                    pi = lax.rem(b, 2)
                    pn = 1 - pi

                    @pl.when(b > 0)
                    def _():
                        pltpu.make_async_copy(
                            rows3.at[pn], acc.at[pl.ds(0, SB)],
                            ssem.at[pn]).wait()

                    @pl.when(b + 1 < NSB)
                    def _():
                        for k in range(KS):
                            pltpu.async_copy(
                                tab_hbm.at[idxq.at[(b + 1) * KS + k]],
                                rows3.at[pn, pl.ds(k * 128, 128)],
                                gsem.at[pn])

                    pltpu.make_async_copy(tab_hbm.at[pl.ds(0, SB)],
                                          rows3.at[pi], gsem.at[pi]).wait()

                    def mul_body(k2, carry3):
                        vr = b * KS + lax.div(k2, 8)
                        vo = lax.rem(k2, 8) * 16
                        vv16 = valq[vr, pl.ds(vo, 16)]
                        for j in range(16):
                            g = k2 * 16 + j
                            vv = jnp.full((32,), vv16[j], jnp.float32
                                          ).astype(jnp.bfloat16)
                            rows3[pi, g, pl.ds(0, 32)] = (
                                rows3[pi, g, pl.ds(0, 32)] * vv)
                            rows3[pi, g, pl.ds(32, 32)] = (
                                rows3[pi, g, pl.ds(32, 32)] * vv)
                        return carry3
                    lax.fori_loop(0, SB // 16, mul_body, 0)

                    for k in range(KS):
                        pltpu.async_copy(rows3.at[pi, pl.ds(k * 128, 128)],
                                         acc.at[rowq.at[b * KS + k]],
                                         ssem.at[pi], add=True)
                    return carry2
                lax.fori_loop(0, NSB, sb_body, 0)

                pltpu.make_async_copy(rows3.at[(NSB - 1) % 2],
                                      acc.at[pl.ds(0, SB)],
                                      ssem.at[(NSB - 1) % 2]).wait()
                return carry
            lax.fori_loop(0, NQ, quarter, 0)

            plsc.subcore_barrier()
            pltpu.sync_copy(acc.at[pl.ds(sid * RPT, RPT)],
                            out_hbm.at[pl.ds(chunk * VP + sid * RPT, RPT)])
            plsc.subcore_barrier()

    return spmv


_spmv_layer1 = _make_spmv(colmul=C, chunkmul=1)
_spmv_layer2 = _make_spmv(colmul=1, chunkmul=VP)


NIDX = B * L
NW = NC * NT
IPW = NIDX // NW
SUB = 80
NSUB = IPW // SUB


def _make_gather():
    mesh = plsc.VectorSubcoreMesh(core_axis_name="c", subcore_axis_name="s")

    @functools.partial(
        pl.kernel,
        mesh=mesh,
        out_type=[
            jax.ShapeDtypeStruct((NIDX, DP), jnp.float32),
            jax.ShapeDtypeStruct((NIDX, DP), jnp.float32),
            jax.ShapeDtypeStruct((C, NIDX, F), jnp.bfloat16),
            jax.ShapeDtypeStruct((C, NIDX, F), jnp.bfloat16),
        ],
        compiler_params=pltpu.CompilerParams(use_tc_tiling_on_sc=False),
        scratch_types=[
            pltpu.VMEM((SUB,), jnp.int32),
            pltpu.VMEM((SUB,), jnp.int32),
            pltpu.VMEM((SUB, DP), jnp.float32),
            pltpu.VMEM((SUB, F), jnp.bfloat16),
            pltpu.SemaphoreType.DMA,
        ],
    )
    def gat(emb_hbm, idxin_hbm, idxsess_hbm, l1_hbm, l2_hbm,
            h_hbm, ge_hbm, gl1_hbm, gl2_hbm,
            ib, ic, rwide, rnarrow, sem):
        cid = lax.axis_index("c")
        sid = lax.axis_index("s")
        wid = sid * NC + cid
        for s in range(NSUB):
            base = wid * IPW + s * SUB
            pltpu.sync_copy(idxin_hbm.at[pl.ds(base, SUB)], ib)
            pltpu.async_copy(emb_hbm.at[ib], rwide, sem).wait()
            pltpu.sync_copy(rwide, h_hbm.at[pl.ds(base, SUB)])
            pltpu.sync_copy(idxsess_hbm.at[pl.ds(base, SUB)], ib)
            pltpu.async_copy(emb_hbm.at[ib], rwide, sem).wait()
            pltpu.sync_copy(rwide, ge_hbm.at[pl.ds(base, SUB)])
            for c in range(C):
                def addoff(i, carry):
                    ic[pl.ds(i * 16, 16)] = ib[pl.ds(i * 16, 16)] + c * VP
                    return carry
                lax.fori_loop(0, SUB // 16, addoff, 0)
                pltpu.async_copy(l1_hbm.at[ic], rnarrow, sem).wait()
                pltpu.sync_copy(rnarrow, gl1_hbm.at[c, pl.ds(base, SUB)])
                pltpu.async_copy(l2_hbm.at[ic], rnarrow, sem).wait()
                pltpu.sync_copy(rnarrow, gl2_hbm.at[c, pl.ds(base, SUB)])

    return gat


_gather_rows = _make_gather()


def _leaky(x):
    return jnp.where(x >= 0, x, ALPHA * x)


GS = 8


def _attn_body(h_ref, em_ref, ge_ref, gl1_ref, gl2_ref, mf_ref, sl_ref, a4_ref,
               out_ref, sess0_ref):
    a4 = a4_ref[...]
    dn = (((1,), (1,)), ((), ()))
    for i in range(GS):
        h = h_ref[i]
        em = em_ref[i]
        ha = jnp.concatenate(
            [h * a4[0], h * a4[1], h * a4[2], h * a4[3]], axis=0)
        e = _leaky(lax.dot_general(ha, h, dn))
        big_neg = jnp.full((L, L), -9e15, jnp.float32)
        al = jnp.where(em == 1, e[0 * L:1 * L], big_neg)
        al = jnp.where(em == 2, e[1 * L:2 * L], al)
        al = jnp.where(em == 3, e[2 * L:3 * L], al)
        al = jnp.where(em == 4, e[3 * L:4 * L], al)
        al = al - jnp.max(al, axis=1, keepdims=True)
        al = jnp.exp(al)
        al = al / jnp.sum(al, axis=1, keepdims=True)
        intra = lax.dot_general(al, h, (((1,), (0,)), ((), ())))
        mf = mf_ref[i]
        seq = (ge_ref[i] + gl1_ref[i] + gl2_ref[i]) * (mf * (1.0 / 3.0))
        out_ref[i] = intra + seq
        seq1 = ge_ref[i] * mf
        sess0_ref[i] = jnp.sum(seq1, axis=0, keepdims=True) / sl_ref[i]


def _sess_body(s0_ref, d_ref, a_ref, out_ref):
    s0 = s0_ref[...]
    da = jnp.dot(d_ref[...], a_ref[...])
    s1 = jnp.dot(da, s0)
    s2 = jnp.dot(da, s1)
    out_ref[...] = (s0 + s1 + s2) * (1.0 / 3.0)


def kernel(inputs, edge_matrix, mask, reversed_sess_item, sess_item, D, A,
           sess_len, embedding, a_0, a_1, a_2, a_3, adj_row, adj_col, adj_val):
    del mask, reversed_sess_item

    emb128 = jnp.pad(embedding.astype(jnp.float32), ((0, 0), (0, DP - DIM)))
    embflat = emb128.astype(jnp.bfloat16).reshape(V * C, F)

    pad = EPAD - E
    padi = jnp.arange(pad, dtype=jnp.int32) % V
    rowp = jnp.concatenate([adj_row.astype(jnp.int32), padi]).reshape(-1, 128)
    colp = jnp.concatenate([adj_col.astype(jnp.int32), padi]).reshape(-1, 128)
    valp = jnp.concatenate([adj_val.astype(jnp.float32),
                            jnp.zeros((pad,), jnp.float32)]).reshape(-1, 128)

    zer = jnp.zeros((VP, F), jnp.bfloat16)
    l1flat = _spmv_layer1(embflat, rowp, colp, valp, zer)
    l2flat = _spmv_layer2(l1flat, rowp, colp, valp, zer)

    idx_in = inputs.astype(jnp.int32).reshape(NIDX)
    si = sess_item.astype(jnp.int32).reshape(NIDX)
    idx_sess = jnp.maximum(si - 1, 0)
    h, ge, gl1, gl2 = _gather_rows(emb128, idx_in, idx_sess, l1flat, l2flat)

    hb = h.reshape(B, L, DP)
    geb = ge.reshape(B, L, DP)
    gl1b = gl1.astype(jnp.float32).transpose(1, 0, 2).reshape(B, L, DP)
    gl2b = gl2.astype(jnp.float32).transpose(1, 0, 2).reshape(B, L, DP)
    mf = (si > 0).astype(jnp.float32).reshape(B, L, 1)
    slr = sess_len.astype(jnp.float32).reshape(B, 1, 1)
    a4 = jnp.concatenate([a_0, a_1, a_2, a_3], axis=1).T.astype(jnp.float32)
    a4 = jnp.pad(a4, ((0, 0), (0, DP - DIM)))
    em = edge_matrix.astype(jnp.int32)

    out, sess0 = pl.pallas_call(
        _attn_body,
        grid=(B // GS,),
        in_specs=[
            pl.BlockSpec((GS, L, DP), lambda b: (b, 0, 0)),
            pl.BlockSpec((GS, L, L), lambda b: (b, 0, 0)),
            pl.BlockSpec((GS, L, DP), lambda b: (b, 0, 0)),
            pl.BlockSpec((GS, L, DP), lambda b: (b, 0, 0)),
            pl.BlockSpec((GS, L, DP), lambda b: (b, 0, 0)),
            pl.BlockSpec((GS, L, 1), lambda b: (b, 0, 0)),
            pl.BlockSpec((GS, 1, 1), lambda b: (b, 0, 0)),
            pl.BlockSpec((4, DP), lambda b: (0, 0)),
        ],
        out_specs=[
            pl.BlockSpec((GS, L, DP), lambda b: (b, 0, 0)),
            pl.BlockSpec((GS, 1, DP), lambda b: (b, 0, 0)),
        ],
        out_shape=[
            jax.ShapeDtypeStruct((B, L, DP), jnp.float32),
            jax.ShapeDtypeStruct((B, 1, DP), jnp.float32),
        ],
    )(hb, em, geb, gl1b, gl2b, mf, slr, a4)

    sess = pl.pallas_call(
        _sess_body,
        out_shape=jax.ShapeDtypeStruct((B, DP), jnp.float32),
    )(sess0.reshape(B, DP), D.astype(jnp.float32), A.astype(jnp.float32))

    return (out[:, :, :DIM], sess[:, :DIM])
